# baseline (speedup 1.0000x reference)
import functools

import jax
import jax.numpy as jnp
from jax import lax
from jax.experimental import pallas as pl
from jax.experimental.pallas import tpu as pltpu
from jax.experimental.pallas import tpu_sc as plsc

B = 1024
K = 1000
D = 512
MARGIN = 5.0

_info = plsc.get_sparse_core_info()
_NC, _NS, _L = _info.num_cores, _info.num_subcores, _info.num_lanes
_NW = _NC * _NS
_BPW = B // _NW


def _sc_pos_body(x_hbm, lab_hbm, c_hbm, out_hbm, idx_v, g_v, x_v, o_v, sem):
    wid = lax.axis_index("s") * _NC + lax.axis_index("c")
    base = wid * _BPW
    pltpu.sync_copy(lab_hbm.at[pl.ds(base, _BPW)], idx_v)
    pltpu.async_copy(c_hbm.at[idx_v], g_v, sem).wait()
    pltpu.sync_copy(x_hbm.at[pl.ds(base, _BPW)], x_v)

    def _row(r, _):
        acc = jnp.zeros((_L,), jnp.float32)
        for k in range(D // _L):
            ga = g_v[r, pl.ds(k * _L, _L)]
            xa = x_v[r, pl.ds(k * _L, _L)]
            acc = acc + ga * (0.5 * ga - xa)
        o_v[r, :] = acc
        return 0

    lax.fori_loop(0, _BPW, _row, 0)
    pltpu.sync_copy(o_v, out_hbm.at[pl.ds(base, _BPW)])


def _sc_pos(x, labels, centers):
    kern = functools.partial(
        pl.kernel,
        mesh=plsc.VectorSubcoreMesh(core_axis_name="c", subcore_axis_name="s"),
        out_type=jax.ShapeDtypeStruct((B, _L), jnp.float32),
        scratch_types=[
            pltpu.VMEM((_BPW,), jnp.int32),
            pltpu.VMEM((_BPW, D), jnp.float32),
            pltpu.VMEM((_BPW, D), jnp.float32),
            pltpu.VMEM((_BPW, _L), jnp.float32),
            pltpu.SemaphoreType.DMA,
        ],
    )(_sc_pos_body)
    return kern(x, labels, centers)


def _tc_neg_body(x_ref, c_ref, lab_ref, out_ref, cch_ref, labc_ref):
    c = c_ref[...]
    cch_ref[...] = 0.5 * jnp.sum(c * c, axis=1)[None, :]
    labc_ref[...] = lab_ref[...].reshape(B, 1)
    prod = jax.lax.dot_general(
        x_ref[...], c, dimension_numbers=(((1,), (1,)), ((), ())),
        preferred_element_type=jnp.float32,
        precision=None)
    d2h = cch_ref[...] - prod
    own = jax.lax.broadcasted_iota(jnp.int32, (B, K), 1) == labc_ref[...]
    out_ref[...] = jnp.min(jnp.where(own, jnp.inf, d2h), axis=1, keepdims=True)


def _tc_combine_body(neg_ref, o2_ref, out_ref):
    pos = jnp.sum(o2_ref[...], axis=1, keepdims=True)
    z = pos - neg_ref[...] + MARGIN
    out_ref[0, 0] = jnp.sum(jnp.log1p(jnp.exp(z))) / B


@jax.jit
def kernel(x, labels, centers):
    labels = labels.astype(jnp.int32)
    o2 = _sc_pos(x, labels, centers)
    neg = pl.pallas_call(
        _tc_neg_body,
        grid=(1,),
        in_specs=[
            pl.BlockSpec((B, D), lambda b: (0, 0)),
            pl.BlockSpec((K, D), lambda b: (0, 0)),
            pl.BlockSpec((B,), lambda b: (0,)),
        ],
        out_specs=pl.BlockSpec((B, 1), lambda b: (0, 0)),
        out_shape=jax.ShapeDtypeStruct((B, 1), jnp.float32),
        scratch_shapes=[
            pltpu.VMEM((1, K), jnp.float32),
            pltpu.VMEM((B, 1), jnp.int32),
        ],
    )(x, centers, labels)
    loss = pl.pallas_call(
        _tc_combine_body,
        grid=(1,),
        in_specs=[
            pl.BlockSpec((B, 1), lambda b: (0, 0)),
            pl.BlockSpec((B, _L), lambda b: (0, 0)),
        ],
        out_specs=pl.BlockSpec(memory_space=pltpu.SMEM),
        out_shape=jax.ShapeDtypeStruct((1, 1), jnp.float32),
    )(neg, o2)
    return loss[0, 0]


# speedup vs baseline: 5.3155x; 5.3155x over previous
import jax
import jax.numpy as jnp
from jax.experimental import pallas as pl
from jax.experimental.pallas import tpu as pltpu

B = 1024
K = 1000
D = 512
MARGIN = 5.0

BB = 1024
NB = B // BB


def _tc_body(x_ref, c_ref, lab_ref, out_ref, cch_ref, labc_ref):
    bb = pl.program_id(0)

    @pl.when(bb == 0)
    def _stage():
        c = c_ref[...]
        cch_ref[...] = 0.5 * jnp.sum(c * c, axis=1)[None, :]
        labc_ref[...] = lab_ref[...].reshape(B, 1)

    x = x_ref[...]
    prod = jax.lax.dot_general(
        x, c_ref[...], dimension_numbers=(((1,), (1,)), ((), ())),
        preferred_element_type=jnp.float32,
        precision=None)
    d2h = cch_ref[...] - prod
    lab = labc_ref[pl.ds(bb * BB, BB), :]
    own = jax.lax.broadcasted_iota(jnp.int32, (BB, K), 1) == lab
    neg = jnp.min(jnp.where(own, jnp.inf, d2h), axis=1, keepdims=True)
    pos = jnp.sum(jnp.where(own, d2h, 0.0), axis=1, keepdims=True)
    z = pos - neg + MARGIN
    partial = jnp.sum(jnp.log1p(jnp.exp(z))) / B

    @pl.when(bb == 0)
    def _first():
        out_ref[0, 0] = partial

    @pl.when(bb > 0)
    def _rest():
        out_ref[0, 0] += partial


@jax.jit
def kernel(x, labels, centers):
    loss = pl.pallas_call(
        _tc_body,
        grid=(NB,),
        in_specs=[
            pl.BlockSpec((BB, D), lambda b: (b, 0)),
            pl.BlockSpec((K, D), lambda b: (0, 0)),
            pl.BlockSpec((B,), lambda b: (0,)),
        ],
        out_specs=pl.BlockSpec(memory_space=pltpu.SMEM),
        out_shape=jax.ShapeDtypeStruct((1, 1), jnp.float32),
        scratch_shapes=[
            pltpu.VMEM((1, K), jnp.float32),
            pltpu.VMEM((B, 1), jnp.int32),
        ],
    )(x, centers, labels.astype(jnp.int32))
    return loss[0, 0]
